# baseline (speedup 1.0000x reference)
import jax
import jax.numpy as jnp
from jax.experimental import pallas as pl

_E = 8
_NEG = -3.0e38


def _fused_body(x_ref, w_ref, b_ref, o_ref):
    xb = x_ref[...]
    m = jnp.dot(xb, w_ref[...], preferred_element_type=jnp.float32)
    m = m + b_ref[...]
    mt = m.T
    logits = mt[:_E, :]
    sig = jax.nn.sigmoid(mt[_E:, :])

    iota = jax.lax.broadcasted_iota(jnp.int32, logits.shape, 0)
    m1 = jnp.max(logits, axis=0, keepdims=True)
    eq1 = logits == m1
    i1 = jnp.min(jnp.where(eq1, iota, _E), axis=0, keepdims=True)
    sel1 = iota == i1
    masked = jnp.where(sel1, _NEG, logits)
    m2 = jnp.max(masked, axis=0, keepdims=True)
    eq2 = masked == m2
    i2 = jnp.min(jnp.where(eq2, iota, _E), axis=0, keepdims=True)
    sel = sel1 | (iota == i2)

    w = jnp.where(sel, jnp.exp(logits - m1), 0.0)
    denom = jnp.sum(w, axis=0, keepdims=True)
    o_ref[...] = jnp.sum(w * sig, axis=0, keepdims=True) / denom


@jax.jit
def kernel(x, Wc, bc, We, be):
    B, S, D = x.shape
    N = B * S
    x2 = x.reshape(N, D)
    W = jnp.concatenate([Wc, We.T], axis=1)
    b = jnp.concatenate([bc, be]).reshape(1, 2 * _E)

    BT = 4096
    out = pl.pallas_call(
        _fused_body,
        grid=(N // BT,),
        in_specs=[
            pl.BlockSpec((BT, D), lambda i: (i, 0)),
            pl.BlockSpec((D, 2 * _E), lambda i: (0, 0)),
            pl.BlockSpec((1, 2 * _E), lambda i: (0, 0)),
        ],
        out_specs=pl.BlockSpec((1, BT), lambda i: (0, i)),
        out_shape=jax.ShapeDtypeStruct((1, N), jnp.float32),
    )(x2, W, b)
    return out.reshape(B, S, 1)
